# baseline (speedup 1.0000x reference)
import functools

import jax
import jax.numpy as jnp
from jax import lax
from jax.experimental import pallas as pl
from jax.experimental.pallas import tpu as pltpu
from jax.experimental.pallas import tpu_sc as plsc

_ALPHA = 0.5
_N_EDGES = 6400000
_NC, _NS, _L = 2, 16, 16
_NW = _NC * _NS
_PER_W = _N_EDGES // _NW
_CHUNK = 20000
_NCHUNK = _PER_W // _CHUNK
_NBUF = 2


@functools.partial(
    pl.kernel,
    out_type=jax.ShapeDtypeStruct((_N_EDGES,), jnp.float32),
    mesh=plsc.VectorSubcoreMesh(core_axis_name="c", subcore_axis_name="s"),
    scratch_types=[
        pltpu.VMEM((_CHUNK,), jnp.float32),
        pltpu.VMEM((_CHUNK,), jnp.float32),
        pltpu.VMEM((_CHUNK,), jnp.float32),
        pltpu.VMEM((_CHUNK,), jnp.float32),
        pltpu.SemaphoreType.DMA,
        pltpu.SemaphoreType.DMA,
        pltpu.SemaphoreType.DMA,
        pltpu.SemaphoreType.DMA,
    ],
)
def _elu_sc(w_hbm, out_hbm, w_v0, w_v1, o_v0, o_v1, si0, si1, so0, so1):
    wid = lax.axis_index("s") * _NC + lax.axis_index("c")
    base = wid * _PER_W
    w_bufs, o_bufs = (w_v0, w_v1), (o_v0, o_v1)
    in_sems, out_sems = (si0, si1), (so0, so1)

    in_d = [None] * _NCHUNK
    out_d = [None] * _NCHUNK
    for ci in range(_NBUF):
        off = base + ci * _CHUNK
        in_d[ci] = pltpu.async_copy(
            w_hbm.at[pl.ds(off, _CHUNK)], w_bufs[ci], in_sems[ci])

    for ci in range(_NCHUNK):
        b = ci % _NBUF
        off = base + ci * _CHUNK
        in_d[ci].wait()
        if ci >= _NBUF:
            out_d[ci - _NBUF].wait()
        w_v, o_v = w_bufs[b], o_bufs[b]

        @plsc.parallel_loop(0, _CHUNK, step=_L, unroll=8)
        def _vec(i):
            x = w_v[pl.ds(i, _L)] * _ALPHA - _ALPHA
            o_v[pl.ds(i, _L)] = (jnp.exp(jnp.minimum(x, 0.0))
                                 + jnp.maximum(x, 0.0))

        out_d[ci] = pltpu.async_copy(
            o_v, out_hbm.at[pl.ds(off, _CHUNK)], out_sems[b])
        nci = ci + _NBUF
        if nci < _NCHUNK:
            noff = base + nci * _CHUNK
            in_d[nci] = pltpu.async_copy(
                w_hbm.at[pl.ds(noff, _CHUNK)], w_bufs[b], in_sems[b])

    out_d[_NCHUNK - 2].wait()
    out_d[_NCHUNK - 1].wait()


_CB = 640000
_NBLK = _N_EDGES // _CB
_NBLK_B = 7


def _copy_b_body(x_ref, o_ref):
    o_ref[...] = x_ref[...]


def _copy_c_body(x_ref, b_ref, w_ref, o_ref):
    del b_ref, w_ref
    o_ref[...] = x_ref[...]


_copy_b = pl.pallas_call(
    _copy_b_body,
    grid=(_NBLK_B,),
    in_specs=[pl.BlockSpec((2, _CB), lambda i: (0, i))],
    out_specs=pl.BlockSpec((2, _CB), lambda i: (0, i)),
    out_shape=jax.ShapeDtypeStruct((2, _N_EDGES), jnp.int32),
)

_copy_c = pl.pallas_call(
    _copy_c_body,
    grid=(_NBLK - _NBLK_B,),
    in_specs=[
        pl.BlockSpec((2, _CB), lambda i: (0, i + _NBLK_B)),
        pl.BlockSpec(memory_space=pl.ANY),
        pl.BlockSpec(memory_space=pl.ANY),
    ],
    out_specs=pl.BlockSpec((2, _CB), lambda i: (0, i + _NBLK_B)),
    out_shape=jax.ShapeDtypeStruct((2, _N_EDGES), jnp.int32),
    input_output_aliases={1: 0},
)


def kernel(edge_index, edge_weight):
    weights = _elu_sc(edge_weight)
    idx_b = _copy_b(edge_index)
    idx_full = _copy_c(edge_index, idx_b, weights)
    return (idx_full, weights)


# speedup vs baseline: 1.0672x; 1.0672x over previous
import functools

import jax
import jax.numpy as jnp
from jax import lax
from jax.experimental import pallas as pl
from jax.experimental.pallas import tpu as pltpu
from jax.experimental.pallas import tpu_sc as plsc

_ALPHA = 0.5
_N_EDGES = 6400000
_NC, _NS, _L = 2, 16, 16
_NW = _NC * _NS
_PER_W = _N_EDGES // _NW
_CHUNK = 20000
_NCHUNK = _PER_W // _CHUNK
_NBUF = 2


@functools.partial(
    pl.kernel,
    out_type=jax.ShapeDtypeStruct((_N_EDGES,), jnp.float32),
    mesh=plsc.VectorSubcoreMesh(core_axis_name="c", subcore_axis_name="s"),
    scratch_types=[
        pltpu.VMEM((_CHUNK,), jnp.float32),
        pltpu.VMEM((_CHUNK,), jnp.float32),
        pltpu.VMEM((_CHUNK,), jnp.float32),
        pltpu.VMEM((_CHUNK,), jnp.float32),
        pltpu.SemaphoreType.DMA,
        pltpu.SemaphoreType.DMA,
        pltpu.SemaphoreType.DMA,
        pltpu.SemaphoreType.DMA,
    ],
)
def _elu_sc(w_hbm, out_hbm, w_v0, w_v1, o_v0, o_v1, si0, si1, so0, so1):
    wid = lax.axis_index("s") * _NC + lax.axis_index("c")
    base = wid * _PER_W
    w_bufs, o_bufs = (w_v0, w_v1), (o_v0, o_v1)
    in_sems, out_sems = (si0, si1), (so0, so1)

    in_d = [None] * _NCHUNK
    out_d = [None] * _NCHUNK
    for ci in range(_NBUF):
        off = base + ci * _CHUNK
        in_d[ci] = pltpu.async_copy(
            w_hbm.at[pl.ds(off, _CHUNK)], w_bufs[ci], in_sems[ci])

    for ci in range(_NCHUNK):
        b = ci % _NBUF
        off = base + ci * _CHUNK
        in_d[ci].wait()
        if ci >= _NBUF:
            out_d[ci - _NBUF].wait()
        w_v, o_v = w_bufs[b], o_bufs[b]

        @plsc.parallel_loop(0, _CHUNK, step=_L, unroll=8)
        def _vec(i):
            x = w_v[pl.ds(i, _L)] * _ALPHA - _ALPHA
            o_v[pl.ds(i, _L)] = (jnp.exp(jnp.minimum(x, 0.0))
                                 + jnp.maximum(x, 0.0))

        out_d[ci] = pltpu.async_copy(
            o_v, out_hbm.at[pl.ds(off, _CHUNK)], out_sems[b])
        nci = ci + _NBUF
        if nci < _NCHUNK:
            noff = base + nci * _CHUNK
            in_d[nci] = pltpu.async_copy(
                w_hbm.at[pl.ds(noff, _CHUNK)], w_bufs[b], in_sems[b])

    out_d[_NCHUNK - 2].wait()
    out_d[_NCHUNK - 1].wait()


_CB = 1600000
_NBLK = _N_EDGES // _CB


def _copy_body(x_ref, o_ref):
    o_ref[...] = x_ref[...]


_tc_copy = pl.pallas_call(
    _copy_body,
    grid=(_NBLK,),
    in_specs=[pl.BlockSpec((2, _CB), lambda i: (0, i))],
    out_specs=pl.BlockSpec((2, _CB), lambda i: (0, i)),
    out_shape=jax.ShapeDtypeStruct((2, _N_EDGES), jnp.int32),
)


def kernel(edge_index, edge_weight):
    return (_tc_copy(edge_index), _elu_sc(edge_weight))


# speedup vs baseline: 1.0711x; 1.0037x over previous
import functools

import jax
import jax.numpy as jnp
from jax import lax
from jax.experimental import pallas as pl
from jax.experimental.pallas import tpu as pltpu
from jax.experimental.pallas import tpu_sc as plsc

_ALPHA = 0.5
_N_EDGES = 6400000
_NC, _NS, _L = 2, 16, 16
_NW = _NC * _NS
_PER_W = _N_EDGES // _NW
_CHUNK = 20000
_NCHUNK = _PER_W // _CHUNK
_NBUF = 2


@functools.partial(
    pl.kernel,
    out_type=jax.ShapeDtypeStruct((_N_EDGES,), jnp.float32),
    mesh=plsc.VectorSubcoreMesh(core_axis_name="c", subcore_axis_name="s"),
    scratch_types=[
        pltpu.VMEM((_CHUNK,), jnp.float32),
        pltpu.VMEM((_CHUNK,), jnp.float32),
        pltpu.VMEM((_CHUNK,), jnp.float32),
        pltpu.VMEM((_CHUNK,), jnp.float32),
        pltpu.SemaphoreType.DMA,
        pltpu.SemaphoreType.DMA,
        pltpu.SemaphoreType.DMA,
        pltpu.SemaphoreType.DMA,
    ],
)
def _elu_sc(w_hbm, out_hbm, w_v0, w_v1, o_v0, o_v1, si0, si1, so0, so1):
    wid = lax.axis_index("s") * _NC + lax.axis_index("c")
    base = wid * _PER_W
    w_bufs, o_bufs = (w_v0, w_v1), (o_v0, o_v1)
    in_sems, out_sems = (si0, si1), (so0, so1)

    for b in range(_NBUF):
        off = base + b * _CHUNK
        pltpu.async_copy(w_hbm.at[pl.ds(off, _CHUNK)], w_bufs[b], in_sems[b])

    @pl.loop(0, _NCHUNK, step=_NBUF)
    def _chunks(g):
        for b in range(_NBUF):
            ci = g + b
            off = base + ci * _CHUNK
            w_v, o_v = w_bufs[b], o_bufs[b]
            pltpu.make_async_copy(
                w_hbm.at[pl.ds(off, _CHUNK)], w_v, in_sems[b]).wait()

            @pl.when(ci >= _NBUF)
            def _():
                pltpu.make_async_copy(
                    o_v, out_hbm.at[pl.ds(off - _NBUF * _CHUNK, _CHUNK)],
                    out_sems[b]).wait()

            @plsc.parallel_loop(0, _CHUNK, step=_L, unroll=8)
            def _vec(i):
                x = w_v[pl.ds(i, _L)] * _ALPHA - _ALPHA
                o_v[pl.ds(i, _L)] = (jnp.exp(jnp.minimum(x, 0.0))
                                     + jnp.maximum(x, 0.0))

            pltpu.async_copy(
                o_v, out_hbm.at[pl.ds(off, _CHUNK)], out_sems[b])

            @pl.when(ci + _NBUF < _NCHUNK)
            def _():
                pltpu.async_copy(
                    w_hbm.at[pl.ds(off + _NBUF * _CHUNK, _CHUNK)],
                    w_v, in_sems[b])

    for b in range(_NBUF):
        off = base + (_NCHUNK - _NBUF + b) * _CHUNK
        pltpu.make_async_copy(
            o_bufs[b], out_hbm.at[pl.ds(off, _CHUNK)], out_sems[b]).wait()


_CB = 1600000
_NBLK = _N_EDGES // _CB


def _copy_body(x_ref, o_ref):
    o_ref[...] = x_ref[...]


_tc_copy = pl.pallas_call(
    _copy_body,
    grid=(_NBLK,),
    in_specs=[pl.BlockSpec((2, _CB), lambda i: (0, i))],
    out_specs=pl.BlockSpec((2, _CB), lambda i: (0, i)),
    out_shape=jax.ShapeDtypeStruct((2, _N_EDGES), jnp.int32),
)


def kernel(edge_index, edge_weight):
    return (_tc_copy(edge_index), _elu_sc(edge_weight))
